# baseline (speedup 1.0000x reference)
import jax
import jax.numpy as jnp
from jax import lax
from jax.experimental import pallas as pl
from jax.experimental.pallas import tpu as pltpu
from jax.experimental.pallas import tpu_sc as plsc

ROWS = 16 * 2048
COLS = 512

R_SC = 8192
R_TC = ROWS - R_SC
BLK = 2048
GRID = R_TC // BLK

NW = 32
N_SC = R_SC * COLS
W = N_SC // NW
CH = 16384
NCH = W // CH

_LN2 = 0.6931471805599453
_LN2_126 = 126.0 * _LN2
_SCALE = 2.0 ** 126
_SQRT2 = 1.4142135623730951


def _tc_kernel(hard_ref, soft_ref, out_ref, acc_ref, cnt_ref):
    i = pl.program_id(0)

    @pl.when(i == 0)
    def _init():
        acc_ref[...] = jnp.zeros_like(acc_ref)
        cnt_ref[...] = jnp.zeros_like(cnt_ref)

    zero = jnp.zeros((8, COLS), jnp.float32)
    zeroi = jnp.zeros((8, COLS), jnp.int32)
    accs = [zero]
    cnts = [zeroi]
    for k in range(BLK // 32):
        xs, hs = [], []
        for q in range(4):
            sl = pl.ds(k * 32 + q * 8, 8)
            h = hard_ref[sl]
            xs.append(jnp.where(h == 1, jnp.maximum(soft_ref[sl], 1e-12), 1.0))
            hs.append(h)
        p = ((xs[0] * xs[1]) * _SCALE) * (xs[2] * xs[3])
        accs[0] += jnp.log(p) - _LN2_126
        cnts[0] += (hs[0] + hs[1]) + (hs[2] + hs[3])

    acc_ref[...] += accs[0]
    cnt_ref[...] += cnts[0].astype(jnp.float32)

    @pl.when(i == GRID - 1)
    def _fini():
        out_ref[0, 0] = -jnp.sum(acc_ref[...])
        out_ref[0, 1] = jnp.sum(cnt_ref[...])


def _tc_partial(hard, soft):
    return pl.pallas_call(
        _tc_kernel,
        grid=(GRID,),
        in_specs=[
            pl.BlockSpec((BLK, COLS), lambda i: (i, 0)),
            pl.BlockSpec((BLK, COLS), lambda i: (i, 0)),
        ],
        out_specs=pl.BlockSpec(memory_space=pltpu.SMEM),
        out_shape=jax.ShapeDtypeStruct((1, 2), jnp.float32),
        scratch_shapes=[
            pltpu.VMEM((8, COLS), jnp.float32),
            pltpu.VMEM((8, COLS), jnp.float32),
        ],
        compiler_params=pltpu.CompilerParams(
            dimension_semantics=("arbitrary",),
        ),
    )(hard, soft)


def _sc_chunk_sum(hbuf, sbuf, acc, cnt):
    def grp(j, carry):
        acc, cnt = carry
        b = j * 64
        hs = [hbuf[pl.ds(b + 16 * t, 16)] for t in range(4)]
        ss = [sbuf[pl.ds(b + 16 * t, 16)] for t in range(4)]
        xs = [jnp.where(h == 1, jnp.maximum(s, 1e-12), 1.0)
              for h, s in zip(hs, ss)]
        p = ((xs[0] * xs[1]) * _SCALE) * (xs[2] * xs[3])
        bits = lax.bitcast_convert_type(p, jnp.int32)
        e = (bits >> 23) - (127 + 126)
        m = lax.bitcast_convert_type(
            (bits & 0x007FFFFF) | 0x3F800000, jnp.float32)
        big = m > _SQRT2
        m = jnp.where(big, m * 0.5, m)
        e = jnp.where(big, e + 1, e)
        z = (m - 1.0) / (m + 1.0)
        z2 = z * z
        poly = 2.0 + z2 * (2.0 / 3.0 + z2 * (2.0 / 5.0 + z2 * (2.0 / 7.0)))
        l = z * poly + e.astype(jnp.float32) * _LN2
        return acc + l, cnt + ((hs[0] + hs[1]) + (hs[2] + hs[3]))

    return lax.fori_loop(0, CH // 64, grp, (acc, cnt))


def _sc_partial(hard_flat, soft_flat):
    mesh = plsc.VectorSubcoreMesh(core_axis_name="c", subcore_axis_name="s")

    @pl.kernel(
        mesh=mesh,
        out_type=[
            jax.ShapeDtypeStruct((NW, 16), jnp.float32),
            jax.ShapeDtypeStruct((NW, 16), jnp.float32),
        ],
        scratch_types=[
            pltpu.VMEM((CH,), jnp.int32),
            pltpu.VMEM((CH,), jnp.float32),
            pltpu.VMEM((CH,), jnp.int32),
            pltpu.VMEM((CH,), jnp.float32),
            pltpu.VMEM((16,), jnp.float32),
            pltpu.VMEM((16,), jnp.float32),
            pltpu.SemaphoreType.DMA,
            pltpu.SemaphoreType.DMA,
        ],
    )
    def sc_kernel(hard_hbm, soft_hbm, sum_out, cnt_out,
                  hbuf0, sbuf0, hbuf1, sbuf1, sstage, cstage, sem0, sem1):
        cid = lax.axis_index("c")
        sid = lax.axis_index("s")
        wid = sid * 2 + cid
        base = wid * W

        hbufs = [hbuf0, hbuf1]
        sbufs = [sbuf0, sbuf1]
        sems = [sem0, sem1]

        def issue(g):
            slot = g % 2
            off = base + g * CH
            ch = pltpu.async_copy(hard_hbm.at[pl.ds(off, CH)],
                                  hbufs[slot], sems[slot])
            cs = pltpu.async_copy(soft_hbm.at[pl.ds(off, CH)],
                                  sbufs[slot], sems[slot])
            return ch, cs

        acc = jnp.zeros((16,), jnp.float32)
        cnt = jnp.zeros((16,), jnp.int32)
        pending = issue(0)
        for g in range(NCH):
            slot = g % 2
            nxt = issue(g + 1) if g + 1 < NCH else None
            pending[0].wait()
            pending[1].wait()
            acc, cnt = _sc_chunk_sum(hbufs[slot], sbufs[slot], acc, cnt)
            pending = nxt

        sstage[...] = acc
        cstage[...] = cnt.astype(jnp.float32)
        pltpu.sync_copy(sstage, sum_out.at[wid])
        pltpu.sync_copy(cstage, cnt_out.at[wid])

    return sc_kernel(hard_flat, soft_flat)


def kernel(hard_attention, soft_attention):
    hard = hard_attention.reshape(ROWS, COLS)
    soft = soft_attention.reshape(ROWS, COLS)

    sc_sums, sc_cnts = _sc_partial(
        hard[R_TC:].reshape(-1), soft[R_TC:].reshape(-1))
    tc_part = _tc_partial(hard[:R_TC], soft[:R_TC])

    total = tc_part[0, 0] + (-jnp.sum(sc_sums))
    count = tc_part[0, 1] + jnp.sum(sc_cnts)
    return total / count


# speedup vs baseline: 1.0655x; 1.0655x over previous
import jax
import jax.numpy as jnp
from jax import lax
from jax.experimental import pallas as pl
from jax.experimental.pallas import tpu as pltpu
from jax.experimental.pallas import tpu_sc as plsc

ROWS = 16 * 2048
COLS = 512

R_SC = 8192
R_TC = ROWS - R_SC
BLK = 2048
GRID = R_TC // BLK

NW = 32
N_SC = R_SC * COLS
W = N_SC // NW
CH = 16384
NCH = W // CH
SC_BASE = R_TC * COLS

_LN2 = 0.6931471805599453
_LN2_126 = 126.0 * _LN2
_SCALE = 2.0 ** 126
_SQRT2 = 1.4142135623730951


def _tc_kernel(hard_ref, soft_ref, out_ref, acc_ref, cnt_ref):
    i = pl.program_id(0)

    @pl.when(i == 0)
    def _init():
        acc_ref[...] = jnp.zeros_like(acc_ref)
        cnt_ref[...] = jnp.zeros_like(cnt_ref)

    acc = jnp.zeros((8, COLS), jnp.float32)
    cnt = jnp.zeros((8, COLS), jnp.int32)
    for k in range(BLK // 32):
        xs, hs = [], []
        for q in range(4):
            sl = pl.ds(k * 32 + q * 8, 8)
            h = hard_ref[sl]
            xs.append(jnp.where(h == 1, jnp.maximum(soft_ref[sl], 1e-12), 1.0))
            hs.append(h)
        p = ((xs[0] * xs[1]) * _SCALE) * (xs[2] * xs[3])
        acc += jnp.log(p) - _LN2_126
        cnt += (hs[0] + hs[1]) + (hs[2] + hs[3])

    acc_ref[...] += acc
    cnt_ref[...] += cnt.astype(jnp.float32)

    @pl.when(i == GRID - 1)
    def _fini():
        out_ref[0, 0] = -jnp.sum(acc_ref[...])
        out_ref[0, 1] = jnp.sum(cnt_ref[...])


def _tc_partial(hard, soft):
    return pl.pallas_call(
        _tc_kernel,
        grid=(GRID,),
        in_specs=[
            pl.BlockSpec((BLK, COLS), lambda i: (i, 0)),
            pl.BlockSpec((BLK, COLS), lambda i: (i, 0)),
        ],
        out_specs=pl.BlockSpec(memory_space=pltpu.SMEM),
        out_shape=jax.ShapeDtypeStruct((1, 2), jnp.float32),
        scratch_shapes=[
            pltpu.VMEM((8, COLS), jnp.float32),
            pltpu.VMEM((8, COLS), jnp.float32),
        ],
        compiler_params=pltpu.CompilerParams(
            dimension_semantics=("arbitrary",),
        ),
    )(hard, soft)


def _group(hbuf, sbuf, b):
    hs = [hbuf[pl.ds(b + 16 * t, 16)] for t in range(4)]
    ss = [sbuf[pl.ds(b + 16 * t, 16)] for t in range(4)]
    xs = [jnp.where(h == 1, jnp.maximum(s, 1e-12), 1.0)
          for h, s in zip(hs, ss)]
    p = ((xs[0] * xs[1]) * _SCALE) * (xs[2] * xs[3])
    bits = lax.bitcast_convert_type(p, jnp.int32)
    e = (bits >> 23) - (127 + 126)
    m = lax.bitcast_convert_type(
        (bits & 0x007FFFFF) | 0x3F800000, jnp.float32)
    big = m > _SQRT2
    m = jnp.where(big, m * 0.5, m)
    e = jnp.where(big, e + 1, e)
    z = (m - 1.0) / (m + 1.0)
    z2 = z * z
    poly = 2.0 + z2 * (2.0 / 3.0 + z2 * (2.0 / 5.0 + z2 * (2.0 / 7.0)))
    l = z * poly + e.astype(jnp.float32) * _LN2
    return l, (hs[0] + hs[1]) + (hs[2] + hs[3])


_UNROLL = 4


def _sc_chunk_sum(hbuf, sbuf, accs, cnts):
    def step(j, carry):
        a, c = list(carry[0]), list(carry[1])
        for u in range(_UNROLL):
            l, cc = _group(hbuf, sbuf, j * (64 * _UNROLL) + u * 64)
            a[u] = a[u] + l
            c[u] = c[u] + cc
        return tuple(a), tuple(c)

    return lax.fori_loop(0, CH // (64 * _UNROLL), step, (accs, cnts))


def _sc_partial(hard_flat, soft_flat):
    mesh = plsc.VectorSubcoreMesh(core_axis_name="c", subcore_axis_name="s")

    @pl.kernel(
        mesh=mesh,
        out_type=[
            jax.ShapeDtypeStruct((NW, 16), jnp.float32),
            jax.ShapeDtypeStruct((NW, 16), jnp.float32),
        ],
        scratch_types=[
            pltpu.VMEM((CH,), jnp.int32),
            pltpu.VMEM((CH,), jnp.float32),
            pltpu.VMEM((CH,), jnp.int32),
            pltpu.VMEM((CH,), jnp.float32),
            pltpu.VMEM((16,), jnp.float32),
            pltpu.VMEM((16,), jnp.float32),
            pltpu.SemaphoreType.DMA,
            pltpu.SemaphoreType.DMA,
        ],
    )
    def sc_kernel(hard_hbm, soft_hbm, sum_out, cnt_out,
                  hbuf0, sbuf0, hbuf1, sbuf1, sstage, cstage, sem0, sem1):
        cid = lax.axis_index("c")
        sid = lax.axis_index("s")
        wid = sid * 2 + cid
        base = SC_BASE + wid * W

        hbufs = [hbuf0, hbuf1]
        sbufs = [sbuf0, sbuf1]
        sems = [sem0, sem1]

        def issue(g):
            slot = g % 2
            off = base + g * CH
            ch = pltpu.async_copy(hard_hbm.at[pl.ds(off, CH)],
                                  hbufs[slot], sems[slot])
            cs = pltpu.async_copy(soft_hbm.at[pl.ds(off, CH)],
                                  sbufs[slot], sems[slot])
            return ch, cs

        zero = jnp.zeros((16,), jnp.float32)
        zeroi = jnp.zeros((16,), jnp.int32)
        accs = (zero,) * _UNROLL
        cnts = (zeroi,) * _UNROLL
        pending = issue(0)
        for g in range(NCH):
            slot = g % 2
            nxt = issue(g + 1) if g + 1 < NCH else None
            pending[0].wait()
            pending[1].wait()
            accs, cnts = _sc_chunk_sum(hbufs[slot], sbufs[slot], accs, cnts)
            pending = nxt

        sstage[...] = (accs[0] + accs[1]) + (accs[2] + accs[3])
        cstage[...] = ((cnts[0] + cnts[1]) + (cnts[2] + cnts[3])).astype(
            jnp.float32)
        pltpu.sync_copy(sstage, sum_out.at[wid])
        pltpu.sync_copy(cstage, cnt_out.at[wid])

    return sc_kernel(hard_flat, soft_flat)


def kernel(hard_attention, soft_attention):
    hard = hard_attention.reshape(ROWS, COLS)
    soft = soft_attention.reshape(ROWS, COLS)

    sc_sums, sc_cnts = _sc_partial(
        hard_attention.reshape(-1), soft_attention.reshape(-1))
    tc_part = _tc_partial(hard, soft)

    total = tc_part[0, 0] + (-jnp.sum(sc_sums))
    count = tc_part[0, 1] + jnp.sum(sc_cnts)
    return total / count


# speedup vs baseline: 2.6174x; 2.4565x over previous
import jax
import jax.numpy as jnp
from jax import lax
from jax.experimental import pallas as pl
from jax.experimental.pallas import tpu as pltpu
from jax.experimental.pallas import tpu_sc as plsc

ROWS = 16 * 2048
COLS = 512

R_SC = 8192
R_TC = ROWS - R_SC
BLK = 2048
GRID = R_TC // BLK

NW = 32
RPW = R_SC // NW
CROWS = 32
NCH = RPW // CROWS

_LN2 = 0.6931471805599453
_LN2_126 = 126.0 * _LN2
_SCALE = 2.0 ** 126
_SQRT2 = 1.4142135623730951


def _tc_kernel(hard_ref, soft_ref, out_ref, acc_ref, cnt_ref):
    i = pl.program_id(0)

    @pl.when(i == 0)
    def _init():
        acc_ref[...] = jnp.zeros_like(acc_ref)
        cnt_ref[...] = jnp.zeros_like(cnt_ref)

    acc = jnp.zeros((8, COLS), jnp.float32)
    cnt = jnp.zeros((8, COLS), jnp.int32)
    for k in range(BLK // 32):
        xs, hs = [], []
        for q in range(4):
            sl = pl.ds(k * 32 + q * 8, 8)
            h = hard_ref[sl]
            xs.append(jnp.where(h == 1, jnp.maximum(soft_ref[sl], 1e-12), 1.0))
            hs.append(h)
        p = ((xs[0] * xs[1]) * _SCALE) * (xs[2] * xs[3])
        acc += jnp.log(p) - _LN2_126
        cnt += (hs[0] + hs[1]) + (hs[2] + hs[3])

    acc_ref[...] += acc
    cnt_ref[...] += cnt.astype(jnp.float32)

    @pl.when(i == GRID - 1)
    def _fini():
        out_ref[0, 0] = -jnp.sum(acc_ref[...])
        out_ref[0, 1] = jnp.sum(cnt_ref[...])


def _tc_partial(hard, soft):
    return pl.pallas_call(
        _tc_kernel,
        grid=(GRID,),
        in_specs=[
            pl.BlockSpec((BLK, COLS), lambda i: (i, 0)),
            pl.BlockSpec((BLK, COLS), lambda i: (i, 0)),
        ],
        out_specs=pl.BlockSpec(memory_space=pltpu.SMEM),
        out_shape=jax.ShapeDtypeStruct((1, 2), jnp.float32),
        scratch_shapes=[
            pltpu.VMEM((8, COLS), jnp.float32),
            pltpu.VMEM((8, COLS), jnp.float32),
        ],
        compiler_params=pltpu.CompilerParams(
            dimension_semantics=("arbitrary",),
        ),
    )(hard, soft)


def _group(hbuf, sbuf, row, b):
    hs = [hbuf[row, pl.ds(b + 16 * t, 16)] for t in range(4)]
    ss = [sbuf[row, pl.ds(b + 16 * t, 16)] for t in range(4)]
    xs = [jnp.where(h == 1, jnp.maximum(s, 1e-12), 1.0)
          for h, s in zip(hs, ss)]
    p = ((xs[0] * xs[1]) * _SCALE) * (xs[2] * xs[3])
    bits = lax.bitcast_convert_type(p, jnp.int32)
    e = (bits >> 23) - (127 + 126)
    m = lax.bitcast_convert_type(
        (bits & 0x007FFFFF) | 0x3F800000, jnp.float32)
    big = m > _SQRT2
    m = jnp.where(big, m * 0.5, m)
    e = jnp.where(big, e + 1, e)
    z = (m - 1.0) / (m + 1.0)
    z2 = z * z
    poly = 2.0 + z2 * (2.0 / 3.0 + z2 * (2.0 / 5.0 + z2 * (2.0 / 7.0)))
    l = z * poly + e.astype(jnp.float32) * _LN2
    return l, (hs[0] + hs[1]) + (hs[2] + hs[3])


_UNROLL = 4


def _sc_chunk_sum(hbuf, sbuf, accs, cnts):
    def row(j, carry):
        a, c = list(carry[0]), list(carry[1])
        for u in range(COLS // 64):
            l, cc = _group(hbuf, sbuf, j, u * 64)
            a[u % _UNROLL] = a[u % _UNROLL] + l
            c[u % _UNROLL] = c[u % _UNROLL] + cc
        return tuple(a), tuple(c)

    return lax.fori_loop(0, CROWS, row, (accs, cnts))


def _sc_partial(hard_flat, soft_flat):
    mesh = plsc.VectorSubcoreMesh(core_axis_name="c", subcore_axis_name="s")

    @pl.kernel(
        mesh=mesh,
        out_type=[
            jax.ShapeDtypeStruct((NW, 16), jnp.float32),
            jax.ShapeDtypeStruct((NW, 16), jnp.float32),
        ],
        scratch_types=[
            pltpu.VMEM((CROWS, COLS), jnp.int32),
            pltpu.VMEM((CROWS, COLS), jnp.float32),
            pltpu.VMEM((CROWS, COLS), jnp.int32),
            pltpu.VMEM((CROWS, COLS), jnp.float32),
            pltpu.VMEM((16,), jnp.float32),
            pltpu.VMEM((16,), jnp.float32),
            pltpu.SemaphoreType.DMA,
            pltpu.SemaphoreType.DMA,
        ],
    )
    def sc_kernel(hard_hbm, soft_hbm, sum_out, cnt_out,
                  hbuf0, sbuf0, hbuf1, sbuf1, sstage, cstage, sem0, sem1):
        cid = lax.axis_index("c")
        sid = lax.axis_index("s")
        wid = sid * 2 + cid
        base = R_TC + wid * RPW

        hbufs = [hbuf0, hbuf1]
        sbufs = [sbuf0, sbuf1]
        sems = [sem0, sem1]

        def issue(g):
            slot = g % 2
            row0 = base + g * CROWS
            ch = pltpu.async_copy(hard_hbm.at[pl.ds(row0, CROWS)],
                                  hbufs[slot], sems[slot])
            cs = pltpu.async_copy(soft_hbm.at[pl.ds(row0, CROWS)],
                                  sbufs[slot], sems[slot])
            return ch, cs

        zero = jnp.zeros((16,), jnp.float32)
        zeroi = jnp.zeros((16,), jnp.int32)
        accs = (zero,) * _UNROLL
        cnts = (zeroi,) * _UNROLL
        pending = issue(0)
        for g in range(NCH):
            slot = g % 2
            nxt = issue(g + 1) if g + 1 < NCH else None
            pending[0].wait()
            pending[1].wait()
            accs, cnts = _sc_chunk_sum(hbufs[slot], sbufs[slot], accs, cnts)
            pending = nxt

        sstage[...] = (accs[0] + accs[1]) + (accs[2] + accs[3])
        cstage[...] = ((cnts[0] + cnts[1]) + (cnts[2] + cnts[3])).astype(
            jnp.float32)
        pltpu.sync_copy(sstage, sum_out.at[wid])
        pltpu.sync_copy(cstage, cnt_out.at[wid])

    return sc_kernel(hard_flat, soft_flat)


def kernel(hard_attention, soft_attention):
    hard = hard_attention.reshape(ROWS, COLS)
    soft = soft_attention.reshape(ROWS, COLS)

    sc_sums, sc_cnts = _sc_partial(hard, soft)
    tc_part = _tc_partial(hard, soft)

    total = tc_part[0, 0] + (-jnp.sum(sc_sums))
    count = tc_part[0, 1] + jnp.sum(sc_cnts)
    return total / count


# speedup vs baseline: 4.1438x; 1.5832x over previous
import jax
import jax.numpy as jnp
from jax.experimental import pallas as pl
from jax.experimental.pallas import tpu as pltpu

ROWS = 16 * 2048
COLS = 512
BLK = 2048
GRID = ROWS // BLK
H = BLK // 4

_LN2_126 = 126.0 * 0.6931471805599453
_SCALE = 2.0 ** 126


def _loss_kernel(hard_ref, soft_ref, out_ref, acc_ref, cnt_ref):
    i = pl.program_id(0)

    @pl.when(i == 0)
    def _init():
        acc_ref[...] = jnp.zeros_like(acc_ref)
        cnt_ref[...] = jnp.zeros_like(cnt_ref)

    zero = jnp.zeros((8, COLS), jnp.float32)
    zeroi = jnp.zeros((8, COLS), jnp.int32)
    accs = [zero]
    cnts = [zeroi]
    for k in range(BLK // 32):
        xs, hs = [], []
        for q in range(4):
            sl = pl.ds(k * 32 + q * 8, 8)
            h = hard_ref[sl]
            xs.append(jnp.where(h == 1, jnp.maximum(soft_ref[sl], 1e-12), 1.0))
            hs.append(h)
        p = ((xs[0] * xs[1]) * _SCALE) * (xs[2] * xs[3])
        accs[0] += jnp.log(p) - _LN2_126
        cnts[0] += (hs[0] + hs[1]) + (hs[2] + hs[3])

    acc_ref[...] += accs[0]
    cnt_ref[...] += cnts[0].astype(jnp.float32)

    @pl.when(i == GRID - 1)
    def _fini():
        total = jnp.sum(acc_ref[...])
        count = jnp.sum(cnt_ref[...])
        out_ref[0, 0] = -total / count


def kernel(hard_attention, soft_attention):
    hard = hard_attention.reshape(ROWS, COLS)
    soft = soft_attention.reshape(ROWS, COLS)
    out = pl.pallas_call(
        _loss_kernel,
        grid=(GRID,),
        in_specs=[
            pl.BlockSpec((BLK, COLS), lambda i: (i, 0)),
            pl.BlockSpec((BLK, COLS), lambda i: (i, 0)),
        ],
        out_specs=pl.BlockSpec(memory_space=pltpu.SMEM),
        out_shape=jax.ShapeDtypeStruct((1, 1), jnp.float32),
        scratch_shapes=[
            pltpu.VMEM((8, COLS), jnp.float32),
            pltpu.VMEM((8, COLS), jnp.float32),
        ],
        compiler_params=pltpu.CompilerParams(
            dimension_semantics=("arbitrary",),
        ),
    )(hard, soft)
    return out[0, 0]


# speedup vs baseline: 4.1447x; 1.0002x over previous
import jax
import jax.numpy as jnp
from jax.experimental import pallas as pl
from jax.experimental.pallas import tpu as pltpu

ROWS = 16 * 2048
COLS = 512
HBLK = 1024
GRID = ROWS // (2 * HBLK)

_LN2_126 = 126.0 * 0.6931471805599453
_SCALE = 2.0 ** 126


def _half(hard_ref, soft_ref, acc, cnt):
    for k in range(HBLK // 32):
        xs, hs = [], []
        for q in range(4):
            sl = pl.ds(k * 32 + q * 8, 8)
            h = hard_ref[sl]
            xs.append(jnp.where(h == 1, jnp.maximum(soft_ref[sl], 1e-12), 1.0))
            hs.append(h)
        p = ((xs[0] * xs[1]) * _SCALE) * (xs[2] * xs[3])
        acc += jnp.log(p) - _LN2_126
        cnt += (hs[0] + hs[1]) + (hs[2] + hs[3])
    return acc, cnt


def _loss_kernel(h0_ref, h1_ref, s0_ref, s1_ref, out_ref, acc_ref, cnt_ref):
    i = pl.program_id(0)

    @pl.when(i == 0)
    def _init():
        acc_ref[...] = jnp.zeros_like(acc_ref)
        cnt_ref[...] = jnp.zeros_like(cnt_ref)

    acc = jnp.zeros((8, COLS), jnp.float32)
    cnt = jnp.zeros((8, COLS), jnp.int32)
    acc, cnt = _half(h0_ref, s0_ref, acc, cnt)
    acc, cnt = _half(h1_ref, s1_ref, acc, cnt)

    acc_ref[...] += acc
    cnt_ref[...] += cnt.astype(jnp.float32)

    @pl.when(i == GRID - 1)
    def _fini():
        out_ref[0, 0] = -jnp.sum(acc_ref[...]) / jnp.sum(cnt_ref[...])


def kernel(hard_attention, soft_attention):
    hard = hard_attention.reshape(ROWS, COLS)
    soft = soft_attention.reshape(ROWS, COLS)
    spec0 = pl.BlockSpec((HBLK, COLS), lambda i: (2 * i, 0))
    spec1 = pl.BlockSpec((HBLK, COLS), lambda i: (2 * i + 1, 0))
    out = pl.pallas_call(
        _loss_kernel,
        grid=(GRID,),
        in_specs=[spec0, spec1, spec0, spec1],
        out_specs=pl.BlockSpec(memory_space=pltpu.SMEM),
        out_shape=jax.ShapeDtypeStruct((1, 1), jnp.float32),
        scratch_shapes=[
            pltpu.VMEM((8, COLS), jnp.float32),
            pltpu.VMEM((8, COLS), jnp.float32),
        ],
        compiler_params=pltpu.CompilerParams(
            dimension_semantics=("arbitrary",),
        ),
    )(hard, hard, soft, soft)
    return out[0, 0]
